# baseline (speedup 1.0000x reference)
import functools

import jax
import jax.numpy as jnp
from jax.experimental import pallas as pl

_EPS = 1e-6
_D = 64
_NSYM = 32
_DEPTH = 8
_BLK = 512


def _crsn_body(x_ref, em_ref, ep_ref, wr_ref, wi_ref, lns_ref, lnb_ref,
               mb_ref, hw_ref, hb_ref, cb_ref, adj_ref, dw_ref, db_ref,
               logits_ref, zar_ref, zai_ref, sym_ref, stats_ref):
    i = pl.program_id(0)

    @pl.when(i == 0)
    def _():
        stats_ref[...] = jnp.zeros_like(stats_ref)

    blk = x_ref.shape[0]
    iota_sym = jax.lax.broadcasted_iota(jnp.int32, (blk, _NSYM), 1)

    hi = jax.lax.Precision.HIGHEST
    xb = x_ref[:, 0]
    ohx = (iota_sym == xb[:, None]).astype(jnp.float32)
    em = em_ref[...]
    ep = ep_ref[...]
    zr = jnp.dot(ohx, em * jnp.cos(ep), precision=hi,
                 preferred_element_type=jnp.float32)
    zi = jnp.dot(ohx, em * jnp.sin(ep), precision=hi,
                 preferred_element_type=jnp.float32)

    wr = wr_ref[...]
    wi = wi_ref[...]
    n_mat = jnp.concatenate(
        [jnp.concatenate([wr, -wi], axis=1),
         jnp.concatenate([wi, wr], axis=1)], axis=0)

    cb = cb_ref[...]
    cb_sq = jnp.sum(cb * cb, axis=1)[None, :]
    adjm = adj_ref[...]
    hw = hw_ref[...]
    hb = hb_ref[0, 0]
    lns = lns_ref[...]
    lnb = lnb_ref[...]
    mb = mb_ref[...]

    f32 = jnp.float32
    hp = jnp.zeros((blk, 1), f32)
    rem = jnp.ones((blk, 1), f32)
    zar = jnp.zeros((blk, _D), f32)
    zai = jnp.zeros((blk, _D), f32)
    ponder = jnp.zeros((1, 1), f32)
    vqs = jnp.zeros((1, 1), f32)
    oh_prev = None
    idx = None

    contract1 = (((1,), (1,)), ((), ()))
    for t in range(_DEPTH):
        zf = jnp.concatenate([zr, zi], axis=1)
        nrni = jax.lax.dot_general(zf, n_mat, contract1,
                                   preferred_element_type=f32)
        nr = nrni[:, :_D]
        ni = nrni[:, _D:]

        hyp = jnp.sqrt(nr * nr + ni * ni)
        mag = hyp + _EPS
        mean = jnp.mean(mag, axis=1, keepdims=True)
        dev = mag - mean
        var = jnp.sum(dev * dev, axis=1, keepdims=True) * (1.0 / (_D - 1))
        mn = dev * jax.lax.rsqrt(var + _EPS)
        mn = mn * lns + lnb
        safe = hyp > 0.0
        inv = 1.0 / jnp.where(safe, hyp, 1.0)
        cosv = jnp.where(safe, nr * inv, 1.0)
        sinv = jnp.where(safe, ni * inv, 0.0)
        zr = mn * cosv
        zi = mn * sinv

        mag2 = jnp.sqrt(zr * zr + zi * zi) + _EPS
        sc = jnp.maximum(mag2 + mb, 0.0) / mag2
        zr = zr * sc
        zi = zi * sc

        zf = jnp.concatenate([zr, zi], axis=1)
        s8 = jax.lax.dot_general(zf, hw, contract1,
                                 preferred_element_type=f32)
        p = jax.nn.sigmoid(s8[:, :1] + hb)

        scores = jax.lax.dot_general(zf, cb, contract1,
                                     preferred_element_type=f32)
        dist = cb_sq - 2.0 * scores
        if oh_prev is None:
            dadj = dist
        else:
            gath = jax.lax.dot_general(oh_prev, adjm, contract1,
                                       precision=hi,
                                       preferred_element_type=f32)
            dadj = dist - 0.1 * jax.nn.sigmoid(gath)
        minv = jnp.min(dadj, axis=1, keepdims=True)
        cand = jnp.where(dadj <= minv, iota_sym, _NSYM)
        idx = jnp.min(cand, axis=1)
        oh = (iota_sym == idx[:, None]).astype(f32)

        zq = jnp.dot(oh, cb, precision=hi, preferred_element_type=f32)
        dq = zq - zf
        vq_row = jnp.sum(dq * dq, axis=1, keepdims=True)
        zr = 0.7 * zr + 0.3 * zq[:, :_D]
        zi = 0.7 * zi + 0.3 * zq[:, _D:]

        still = (hp < 0.99).astype(f32)
        p_eff = rem if t == _DEPTH - 1 else p * still
        zar = zar + p_eff * zr
        zai = zai + p_eff * zi
        hp = hp + p_eff
        rem = rem - p_eff
        ponder = ponder + jnp.sum(still, axis=(0, 1), keepdims=True)
        vqs = vqs + jnp.sum(vq_row, axis=(0, 1), keepdims=True)
        oh_prev = oh

    feats = jnp.concatenate([zar, zai], axis=1)
    logits = jax.lax.dot_general(feats, dw_ref[...], contract1,
                                 preferred_element_type=f32) + db_ref[...]
    logits_ref[...] = logits
    zar_ref[...] = zar
    zai_ref[...] = zai
    sym_ref[...] = idx[:, None]
    stats_ref[...] += jnp.concatenate([ponder, vqs], axis=1)


@functools.partial(jax.jit, static_argnames=("interpret",))
def _run(x, emb_mag, emb_phase, Wr, Wi, ln_scale, ln_shift, mod_bias,
         halt_W, halt_b, codebook, adj, dec_W, dec_b, interpret=False):
    batch = x.shape[0]
    vocab, d = emb_mag.shape
    nb = batch // _BLK

    x2 = x.astype(jnp.int32).reshape(batch, 1)
    em_p = jnp.zeros((_NSYM, d), jnp.float32).at[:vocab].set(emb_mag)
    ep_p = jnp.zeros((_NSYM, d), jnp.float32).at[:vocab].set(emb_phase)
    dw_p = jnp.zeros((_NSYM, 2 * d), jnp.float32).at[:dec_W.shape[0]].set(dec_W)
    db_p = jnp.zeros((1, _NSYM), jnp.float32).at[0, :dec_b.shape[0]].set(dec_b)

    full = lambda shape: pl.BlockSpec(shape, lambda i: (0, 0))
    out = pl.pallas_call(
        _crsn_body,
        grid=(nb,),
        in_specs=[
            pl.BlockSpec((_BLK, 1), lambda i: (i, 0)),
            full((_NSYM, d)), full((_NSYM, d)),
            full((d, d)), full((d, d)),
            full((1, d)), full((1, d)), full((1, d)),
            full((8, 2 * d)), full((1, 1)),
            full((_NSYM, 2 * d)), full((_NSYM, _NSYM)),
            full((_NSYM, 2 * d)), full((1, _NSYM)),
        ],
        out_specs=[
            pl.BlockSpec((_BLK, _NSYM), lambda i: (i, 0)),
            pl.BlockSpec((_BLK, d), lambda i: (i, 0)),
            pl.BlockSpec((_BLK, d), lambda i: (i, 0)),
            pl.BlockSpec((_BLK, 1), lambda i: (i, 0)),
            pl.BlockSpec((1, 2), lambda i: (0, 0)),
        ],
        out_shape=[
            jax.ShapeDtypeStruct((batch, _NSYM), jnp.float32),
            jax.ShapeDtypeStruct((batch, d), jnp.float32),
            jax.ShapeDtypeStruct((batch, d), jnp.float32),
            jax.ShapeDtypeStruct((batch, 1), jnp.int32),
            jax.ShapeDtypeStruct((1, 2), jnp.float32),
        ],
        interpret=interpret,
    )(x2, em_p, ep_p, Wr, Wi,
      ln_scale.reshape(1, d), ln_shift.reshape(1, d), mod_bias.reshape(1, d),
      jnp.zeros((8, 2 * d), jnp.float32).at[:1].set(halt_W.reshape(1, 2 * d)),
      halt_b.reshape(1, 1).astype(jnp.float32),
      codebook, adj, dw_p, db_p)

    logits_p, zar, zai, sym2, stats = out
    logits = logits_p[:, :dec_W.shape[0]]
    z_accum = jax.lax.complex(zar, zai)
    sym = sym2[:, 0]
    ponder = stats[0, 0] / batch
    vq_total = stats[0, 1] * (1.25 / (batch * 2 * d))
    return (logits, z_accum, sym, ponder, vq_total)


def kernel(x, emb_mag, emb_phase, Wr, Wi, ln_scale, ln_shift, mod_bias,
           halt_W, halt_b, codebook, adj, dec_W, dec_b):
    return _run(x, emb_mag, emb_phase, Wr, Wi, ln_scale, ln_shift, mod_bias,
                halt_W, halt_b, codebook, adj, dec_W, dec_b)


# speedup vs baseline: 1.1412x; 1.1412x over previous
import functools

import jax
import jax.numpy as jnp
from jax.experimental import pallas as pl

_EPS = 1e-6
_D = 64
_NSYM = 32
_DEPTH = 8
_BLK = 1024


def _crsn_body(x_ref, em_ref, ep_ref, wr_ref, wi_ref, lns_ref, lnb_ref,
               mb_ref, hw_ref, hb_ref, cb_ref, adj_ref, dw_ref, db_ref,
               logits_ref, feats_ref, sym_ref, stats_ref):
    i = pl.program_id(0)

    @pl.when(i == 0)
    def _():
        stats_ref[...] = jnp.zeros_like(stats_ref)

    f32 = jnp.float32
    hi = jax.lax.Precision.HIGHEST
    contract1 = (((1,), (1,)), ((), ()))
    blk = x_ref.shape[0]
    iota_sym = jax.lax.broadcasted_iota(jnp.int32, (blk, _NSYM), 1)

    xb = x_ref[:, 0]
    ohx = (iota_sym == xb[:, None]).astype(f32)
    em = em_ref[...]
    ep = ep_ref[...]
    table = jnp.concatenate([em * jnp.cos(ep), em * jnp.sin(ep)], axis=1)
    zf = jnp.dot(ohx, table, precision=hi, preferred_element_type=f32)

    wr = wr_ref[...]
    wi = wi_ref[...]
    n_mat = jnp.concatenate(
        [jnp.concatenate([wr, -wi], axis=1),
         jnp.concatenate([wi, wr], axis=1)], axis=0)

    cb = cb_ref[...]
    cb_sq = jnp.sum(cb * cb, axis=1)[None, :]
    adjm = adj_ref[...]
    hw = hw_ref[...]
    hb = hb_ref[0, 0]
    lns = lns_ref[...]
    lnb = lnb_ref[...]
    mb = mb_ref[...]
    onezero = jnp.concatenate(
        [jnp.ones((1, _D), f32), jnp.zeros((1, _D), f32)], axis=1)

    hp = jnp.zeros((blk, 1), f32)
    rem = jnp.ones((blk, 1), f32)
    za = jnp.zeros((blk, 2 * _D), f32)
    still_acc = jnp.zeros((blk, 1), f32)
    vq_acc = jnp.zeros((blk, 2 * _D), f32)
    oh_prev = None
    idx = None

    for t in range(_DEPTH):
        nrni = jax.lax.dot_general(zf, n_mat, contract1,
                                   preferred_element_type=f32)
        sq = nrni * nrni
        hyp2 = sq + jnp.concatenate([sq[:, _D:], sq[:, :_D]], axis=1)
        hyp = jnp.sqrt(hyp2)
        mag = hyp + _EPS

        s1 = jnp.sum(mag, axis=1, keepdims=True)
        mean = s1 * (1.0 / (2 * _D))
        dev = mag - mean
        s2 = jnp.sum(dev * dev, axis=1, keepdims=True)
        var = s2 * (1.0 / (2 * (_D - 1)))
        mn = (dev * jax.lax.rsqrt(var + _EPS)) * lns + lnb

        safe = hyp2 > 0.0
        inv = 1.0 / jnp.where(safe, hyp, 1.0)
        cs = jnp.where(safe, nrni * inv, onezero)
        zf = mn * cs

        mag2 = jnp.abs(mn) + _EPS
        sc = jnp.maximum(mag2 + mb, 0.0) / mag2
        zf = zf * sc

        s8 = jax.lax.dot_general(zf, hw, contract1,
                                 preferred_element_type=f32)
        p = jax.nn.sigmoid(s8[:, :1] + hb)

        scores = jax.lax.dot_general(zf, cb, contract1,
                                     preferred_element_type=f32)
        dist = cb_sq - 2.0 * scores
        if oh_prev is None:
            dadj = dist
        else:
            gath = jax.lax.dot_general(oh_prev, adjm, contract1,
                                       precision=hi,
                                       preferred_element_type=f32)
            dadj = dist - 0.1 * jax.nn.sigmoid(gath)
        minv = jnp.min(dadj, axis=1, keepdims=True)
        cand = jnp.where(dadj <= minv, iota_sym, _NSYM)
        idx = jnp.min(cand, axis=1)
        oh = (iota_sym == idx[:, None]).astype(f32)

        zq = jnp.dot(oh, cb, precision=hi, preferred_element_type=f32)
        dq = zq - zf
        vq_acc = vq_acc + dq * dq

        zf = 0.7 * zf + 0.3 * zq

        still = (hp < 0.99).astype(f32)
        p_eff = rem if t == _DEPTH - 1 else p * still
        za = za + p_eff * zf
        hp = hp + p_eff
        rem = rem - p_eff
        still_acc = still_acc + still
        oh_prev = oh

    logits = jax.lax.dot_general(za, dw_ref[...], contract1,
                                 preferred_element_type=f32) + db_ref[...]
    logits_ref[...] = logits
    feats_ref[...] = za
    sym_ref[...] = idx[:, None]
    ponder = jnp.sum(still_acc, axis=(0, 1), keepdims=True)
    vqs = jnp.sum(vq_acc, axis=(0, 1), keepdims=True)
    stats_ref[...] += jnp.concatenate([ponder, vqs], axis=1)


@functools.partial(jax.jit, static_argnames=("interpret",))
def _run(x, emb_mag, emb_phase, Wr, Wi, ln_scale, ln_shift, mod_bias,
         halt_W, halt_b, codebook, adj, dec_W, dec_b, interpret=False):
    batch = x.shape[0]
    vocab, d = emb_mag.shape
    nb = batch // _BLK

    x2 = x.astype(jnp.int32).reshape(batch, 1)
    em_p = jnp.zeros((_NSYM, d), jnp.float32).at[:vocab].set(emb_mag)
    ep_p = jnp.zeros((_NSYM, d), jnp.float32).at[:vocab].set(emb_phase)
    dw_p = jnp.zeros((_NSYM, 2 * d), jnp.float32).at[:dec_W.shape[0]].set(dec_W)
    db_p = jnp.zeros((1, _NSYM), jnp.float32).at[0, :dec_b.shape[0]].set(dec_b)
    lns2 = jnp.concatenate([ln_scale, ln_scale]).reshape(1, 2 * d)
    lnb2 = jnp.concatenate([ln_shift, ln_shift]).reshape(1, 2 * d)
    mb2 = jnp.concatenate([mod_bias, mod_bias]).reshape(1, 2 * d)
    hw8 = jnp.zeros((8, 2 * d), jnp.float32).at[:1].set(halt_W.reshape(1, 2 * d))

    full = lambda shape: pl.BlockSpec(shape, lambda i: (0, 0))
    out = pl.pallas_call(
        _crsn_body,
        grid=(nb,),
        in_specs=[
            pl.BlockSpec((_BLK, 1), lambda i: (i, 0)),
            full((_NSYM, d)), full((_NSYM, d)),
            full((d, d)), full((d, d)),
            full((1, 2 * d)), full((1, 2 * d)), full((1, 2 * d)),
            full((8, 2 * d)), full((1, 1)),
            full((_NSYM, 2 * d)), full((_NSYM, _NSYM)),
            full((_NSYM, 2 * d)), full((1, _NSYM)),
        ],
        out_specs=[
            pl.BlockSpec((_BLK, _NSYM), lambda i: (i, 0)),
            pl.BlockSpec((_BLK, 2 * d), lambda i: (i, 0)),
            pl.BlockSpec((_BLK, 1), lambda i: (i, 0)),
            pl.BlockSpec((1, 2), lambda i: (0, 0)),
        ],
        out_shape=[
            jax.ShapeDtypeStruct((batch, _NSYM), jnp.float32),
            jax.ShapeDtypeStruct((batch, 2 * d), jnp.float32),
            jax.ShapeDtypeStruct((batch, 1), jnp.int32),
            jax.ShapeDtypeStruct((1, 2), jnp.float32),
        ],
        interpret=interpret,
    )(x2, em_p, ep_p, Wr, Wi, lns2, lnb2, mb2, hw8,
      halt_b.reshape(1, 1).astype(jnp.float32), codebook, adj, dw_p, db_p)

    logits_p, feats, sym2, stats = out
    logits = logits_p[:, :dec_W.shape[0]]
    z_accum = jax.lax.complex(feats[:, :d], feats[:, d:])
    sym = sym2[:, 0]
    ponder = stats[0, 0] / batch
    vq_total = stats[0, 1] * (1.25 / (batch * 2 * d))
    return (logits, z_accum, sym, ponder, vq_total)


def kernel(x, emb_mag, emb_phase, Wr, Wi, ln_scale, ln_shift, mod_bias,
           halt_W, halt_b, codebook, adj, dec_W, dec_b):
    return _run(x, emb_mag, emb_phase, Wr, Wi, ln_scale, ln_shift, mod_bias,
                halt_W, halt_b, codebook, adj, dec_W, dec_b)


# speedup vs baseline: 1.1584x; 1.0150x over previous
import functools

import jax
import jax.numpy as jnp
from jax.experimental import pallas as pl

_EPS = 1e-6
_D = 64
_NSYM = 32
_DEPTH = 8
_BLK = 1024


def _crsn_body(x_ref, em_ref, ep_ref, wr_ref, wi_ref, lns_ref, lnb_ref,
               mb_ref, hw_ref, hb_ref, cb_ref, adj_ref, dw_ref, db_ref,
               logits_ref, feats_ref, sym_ref, stats_ref):
    i = pl.program_id(0)

    @pl.when(i == 0)
    def _():
        stats_ref[...] = jnp.zeros_like(stats_ref)

    f32 = jnp.float32
    bf16 = jnp.bfloat16
    contract1 = (((1,), (1,)), ((), ()))
    blk = x_ref.shape[0]
    iota_sym = jax.lax.broadcasted_iota(jnp.int32, (blk, _NSYM), 1)

    def split3(m):
        m1 = m.astype(bf16).astype(f32)
        r1 = m - m1
        m2 = r1.astype(bf16).astype(f32)
        return m1, m2, r1 - m2

    def gather(oh, parts):
        out = jnp.dot(oh, parts[0], preferred_element_type=f32)
        for p in parts[1:]:
            out = out + jnp.dot(oh, p, preferred_element_type=f32)
        return out

    xb = x_ref[:, 0]
    ohx = (iota_sym == xb[:, None]).astype(f32)
    em = em_ref[...]
    ep = ep_ref[...]
    table = jnp.concatenate([em * jnp.cos(ep), em * jnp.sin(ep)], axis=1)
    zf = gather(ohx, split3(table))

    wr = wr_ref[...]
    wi = wi_ref[...]
    n_mat = jnp.concatenate(
        [jnp.concatenate([wr, -wi], axis=1),
         jnp.concatenate([wi, wr], axis=1)], axis=0)

    cb = cb_ref[...]
    cb_sq = jnp.sum(cb * cb, axis=1)[None, :]
    cb_parts = split3(cb)
    adjm = adj_ref[...]
    adj1 = adjm.astype(bf16).astype(f32)
    adj_parts = (adj1, adjm - adj1)
    cbh = jnp.concatenate([cb, hw_ref[...]], axis=0)
    hb = hb_ref[0, 0]
    lns = lns_ref[...]
    lnb = lnb_ref[...]
    mb = mb_ref[...]
    onezero = jnp.concatenate(
        [jnp.ones((1, _D), f32), jnp.zeros((1, _D), f32)], axis=1)

    hp = jnp.zeros((blk, 1), f32)
    rem = jnp.ones((blk, 1), f32)
    za = jnp.zeros((blk, 2 * _D), f32)
    still_acc = jnp.zeros((blk, 1), f32)
    vq_acc = jnp.zeros((blk, 2 * _D), f32)
    oh_prev = None
    idx = None

    for t in range(_DEPTH):
        nrni = jax.lax.dot_general(zf, n_mat, contract1,
                                   preferred_element_type=f32)
        sq = nrni * nrni
        hyp2 = sq + jnp.concatenate([sq[:, _D:], sq[:, :_D]], axis=1)
        safe = hyp2 > 0.0
        inv = jnp.where(safe, jax.lax.rsqrt(hyp2), 0.0)
        mag = hyp2 * inv + _EPS

        s1 = jnp.sum(mag, axis=1, keepdims=True)
        mean = s1 * (1.0 / (2 * _D))
        dev = mag - mean
        s2 = jnp.sum(dev * dev, axis=1, keepdims=True)
        var = s2 * (1.0 / (2 * (_D - 1)))
        mn = (dev * jax.lax.rsqrt(var + _EPS)) * lns + lnb

        cs = jnp.where(safe, nrni * inv, onezero)
        zf = mn * cs

        mag2 = jnp.abs(mn) + _EPS
        sc = jnp.maximum(mag2 + mb, 0.0) / mag2
        zf = zf * sc

        scores_all = jax.lax.dot_general(zf, cbh, contract1,
                                         preferred_element_type=f32)
        p = jax.nn.sigmoid(scores_all[:, _NSYM:_NSYM + 1] + hb)

        dist = cb_sq - 2.0 * scores_all[:, :_NSYM]
        if oh_prev is None:
            dadj = dist
        else:
            dadj = dist - 0.1 * jax.nn.sigmoid(gather(oh_prev, adj_parts))
        minv = jnp.min(dadj, axis=1, keepdims=True)
        cand = jnp.where(dadj <= minv, iota_sym, _NSYM)
        idx = jnp.min(cand, axis=1)
        oh = (iota_sym == idx[:, None]).astype(f32)

        zq = gather(oh, cb_parts)
        dq = zq - zf
        vq_acc = vq_acc + dq * dq

        zf = 0.7 * zf + 0.3 * zq

        still = (hp < 0.99).astype(f32)
        p_eff = rem if t == _DEPTH - 1 else p * still
        za = za + p_eff * zf
        hp = hp + p_eff
        rem = rem - p_eff
        still_acc = still_acc + still
        oh_prev = oh

    logits = jax.lax.dot_general(za, dw_ref[...], contract1,
                                 preferred_element_type=f32) + db_ref[...]
    logits_ref[...] = logits
    feats_ref[...] = za
    sym_ref[...] = idx[:, None]
    ponder = jnp.sum(still_acc, axis=(0, 1), keepdims=True)
    vqs = jnp.sum(vq_acc, axis=(0, 1), keepdims=True)
    stats_ref[...] += jnp.concatenate([ponder, vqs], axis=1)


@functools.partial(jax.jit, static_argnames=("interpret",))
def _run(x, emb_mag, emb_phase, Wr, Wi, ln_scale, ln_shift, mod_bias,
         halt_W, halt_b, codebook, adj, dec_W, dec_b, interpret=False):
    batch = x.shape[0]
    vocab, d = emb_mag.shape
    nb = batch // _BLK

    x2 = x.astype(jnp.int32).reshape(batch, 1)
    em_p = jnp.zeros((_NSYM, d), jnp.float32).at[:vocab].set(emb_mag)
    ep_p = jnp.zeros((_NSYM, d), jnp.float32).at[:vocab].set(emb_phase)
    dw_p = jnp.zeros((_NSYM, 2 * d), jnp.float32).at[:dec_W.shape[0]].set(dec_W)
    db_p = jnp.zeros((1, _NSYM), jnp.float32).at[0, :dec_b.shape[0]].set(dec_b)
    lns2 = jnp.concatenate([ln_scale, ln_scale]).reshape(1, 2 * d)
    lnb2 = jnp.concatenate([ln_shift, ln_shift]).reshape(1, 2 * d)
    mb2 = jnp.concatenate([mod_bias, mod_bias]).reshape(1, 2 * d)
    hw8 = jnp.zeros((8, 2 * d), jnp.float32).at[:1].set(halt_W.reshape(1, 2 * d))

    full = lambda shape: pl.BlockSpec(shape, lambda i: (0, 0))
    out = pl.pallas_call(
        _crsn_body,
        grid=(nb,),
        in_specs=[
            pl.BlockSpec((_BLK, 1), lambda i: (i, 0)),
            full((_NSYM, d)), full((_NSYM, d)),
            full((d, d)), full((d, d)),
            full((1, 2 * d)), full((1, 2 * d)), full((1, 2 * d)),
            full((8, 2 * d)), full((1, 1)),
            full((_NSYM, 2 * d)), full((_NSYM, _NSYM)),
            full((_NSYM, 2 * d)), full((1, _NSYM)),
        ],
        out_specs=[
            pl.BlockSpec((_BLK, _NSYM), lambda i: (i, 0)),
            pl.BlockSpec((_BLK, 2 * d), lambda i: (i, 0)),
            pl.BlockSpec((_BLK, 1), lambda i: (i, 0)),
            pl.BlockSpec((1, 2), lambda i: (0, 0)),
        ],
        out_shape=[
            jax.ShapeDtypeStruct((batch, _NSYM), jnp.float32),
            jax.ShapeDtypeStruct((batch, 2 * d), jnp.float32),
            jax.ShapeDtypeStruct((batch, 1), jnp.int32),
            jax.ShapeDtypeStruct((1, 2), jnp.float32),
        ],
        interpret=interpret,
    )(x2, em_p, ep_p, Wr, Wi, lns2, lnb2, mb2, hw8,
      halt_b.reshape(1, 1).astype(jnp.float32), codebook, adj, dw_p, db_p)

    logits_p, feats, sym2, stats = out
    logits = logits_p[:, :dec_W.shape[0]]
    z_accum = jax.lax.complex(feats[:, :d], feats[:, d:])
    sym = sym2[:, 0]
    ponder = stats[0, 0] / batch
    vq_total = stats[0, 1] * (1.25 / (batch * 2 * d))
    return (logits, z_accum, sym, ponder, vq_total)


def kernel(x, emb_mag, emb_phase, Wr, Wi, ln_scale, ln_shift, mod_bias,
           halt_W, halt_b, codebook, adj, dec_W, dec_b):
    return _run(x, emb_mag, emb_phase, Wr, Wi, ln_scale, ln_shift, mod_bias,
                halt_W, halt_b, codebook, adj, dec_W, dec_b)


# speedup vs baseline: 1.1613x; 1.0025x over previous
import functools

import jax
import jax.numpy as jnp
from jax.experimental import pallas as pl

_EPS = 1e-6
_D = 64
_NSYM = 32
_DEPTH = 8
_BLK = 1024


def _crsn_body(x_ref, em_ref, ep_ref, wr_ref, wi_ref, lns_ref, lnb_ref,
               mb_ref, hw_ref, hb_ref, cb_ref, adj_ref, dw_ref, db_ref,
               logits_ref, feats_ref, sym_ref, stats_ref):
    i = pl.program_id(0)

    @pl.when(i == 0)
    def _():
        stats_ref[...] = jnp.zeros_like(stats_ref)

    f32 = jnp.float32
    bf16 = jnp.bfloat16
    contract1 = (((1,), (1,)), ((), ()))
    blk = x_ref.shape[0]
    iota_sym = jax.lax.broadcasted_iota(jnp.int32, (blk, _NSYM), 1)
    iota_f = iota_sym.astype(jnp.float32)

    def split3(m):
        m1 = m.astype(bf16).astype(f32)
        r1 = m - m1
        m2 = r1.astype(bf16).astype(f32)
        return m1, m2, r1 - m2

    def gather(oh, parts):
        out = jnp.dot(oh, parts[0], preferred_element_type=f32)
        for p in parts[1:]:
            out = out + jnp.dot(oh, p, preferred_element_type=f32)
        return out

    xb = x_ref[:, 0]
    ohx = (iota_sym == xb[:, None]).astype(f32)
    em = em_ref[...]
    ep = ep_ref[...]
    table = jnp.concatenate([em * jnp.cos(ep), em * jnp.sin(ep)], axis=1)
    zf = gather(ohx, split3(table))

    wr = wr_ref[...]
    wi = wi_ref[...]
    n_mat = jnp.concatenate(
        [jnp.concatenate([wr, -wi], axis=1),
         jnp.concatenate([wi, wr], axis=1)], axis=0)

    cb = cb_ref[...]
    cb_sq = jnp.sum(cb * cb, axis=1)[None, :]
    cb_parts = split3(cb)
    adjm = adj_ref[...]
    adj1 = adjm.astype(bf16).astype(f32)
    adj_parts = (adj1, adjm - adj1)
    cbh = jnp.concatenate([cb, hw_ref[...]], axis=0)
    hb = hb_ref[0, 0]
    lns = lns_ref[...]
    lnb = lnb_ref[...]
    mb = mb_ref[...]
    onezero = jnp.concatenate(
        [jnp.ones((1, _D), f32), jnp.zeros((1, _D), f32)], axis=1)

    hp = jnp.zeros((blk, 1), f32)
    rem = jnp.ones((blk, 1), f32)
    za = jnp.zeros((blk, 2 * _D), f32)
    still_acc = jnp.zeros((blk, 1), f32)
    vq_acc = jnp.zeros((blk, 2 * _D), f32)
    oh_prev = None
    idx = None

    for t in range(_DEPTH):
        nrni = jax.lax.dot_general(zf, n_mat, contract1,
                                   preferred_element_type=f32)
        sq = nrni * nrni
        hyp2 = sq + jnp.concatenate([sq[:, _D:], sq[:, :_D]], axis=1)
        safe = hyp2 > 0.0
        inv = jnp.where(safe, jax.lax.rsqrt(hyp2), 0.0)
        mag = hyp2 * inv + _EPS

        s1 = jnp.sum(mag, axis=1, keepdims=True)
        mean = s1 * (1.0 / (2 * _D))
        dev = mag - mean
        s2 = jnp.sum(dev * dev, axis=1, keepdims=True)
        var = s2 * (1.0 / (2 * (_D - 1)))
        mn = (dev * jax.lax.rsqrt(var + _EPS)) * lns + lnb

        cs = jnp.where(safe, nrni * inv, onezero)
        zf = mn * cs

        mag2 = jnp.abs(mn) + _EPS
        sc = jnp.maximum(mag2 + mb, 0.0) / mag2
        zf = zf * sc

        scores_all = jax.lax.dot_general(zf, cbh, contract1,
                                         preferred_element_type=f32)
        p = jax.nn.sigmoid(scores_all[:, _NSYM:_NSYM + 1] + hb)

        dist = cb_sq - 2.0 * scores_all[:, :_NSYM]
        if oh_prev is None:
            dadj = dist
        else:
            dadj = dist - 0.1 * jax.nn.sigmoid(gather(oh_prev, adj_parts))
        minv = jnp.min(dadj, axis=1, keepdims=True)
        cand = jnp.where(dadj <= minv, iota_f, float(_NSYM))
        idx = jnp.min(cand, axis=1, keepdims=True)
        oh = (iota_f == idx).astype(f32)

        zq = gather(oh, cb_parts)
        dq = zq - zf
        vq_acc = vq_acc + dq * dq

        zf = 0.7 * zf + 0.3 * zq

        still = (hp < 0.99).astype(f32)
        p_eff = rem if t == _DEPTH - 1 else p * still
        za = za + p_eff * zf
        hp = hp + p_eff
        rem = rem - p_eff
        still_acc = still_acc + still
        oh_prev = oh

    logits = jax.lax.dot_general(za, dw_ref[...], contract1,
                                 preferred_element_type=f32) + db_ref[...]
    logits_ref[...] = logits
    feats_ref[...] = za
    sym_ref[...] = idx.astype(jnp.int32)
    ponder = jnp.sum(still_acc, axis=(0, 1), keepdims=True)
    vqs = jnp.sum(vq_acc, axis=(0, 1), keepdims=True)
    stats_ref[...] += jnp.concatenate([ponder, vqs], axis=1)


@functools.partial(jax.jit, static_argnames=("interpret",))
def _run(x, emb_mag, emb_phase, Wr, Wi, ln_scale, ln_shift, mod_bias,
         halt_W, halt_b, codebook, adj, dec_W, dec_b, interpret=False):
    batch = x.shape[0]
    vocab, d = emb_mag.shape
    nb = batch // _BLK

    x2 = x.astype(jnp.int32).reshape(batch, 1)
    em_p = jnp.zeros((_NSYM, d), jnp.float32).at[:vocab].set(emb_mag)
    ep_p = jnp.zeros((_NSYM, d), jnp.float32).at[:vocab].set(emb_phase)
    dw_p = jnp.zeros((_NSYM, 2 * d), jnp.float32).at[:dec_W.shape[0]].set(dec_W)
    db_p = jnp.zeros((1, _NSYM), jnp.float32).at[0, :dec_b.shape[0]].set(dec_b)
    lns2 = jnp.concatenate([ln_scale, ln_scale]).reshape(1, 2 * d)
    lnb2 = jnp.concatenate([ln_shift, ln_shift]).reshape(1, 2 * d)
    mb2 = jnp.concatenate([mod_bias, mod_bias]).reshape(1, 2 * d)
    hw8 = jnp.zeros((8, 2 * d), jnp.float32).at[:1].set(halt_W.reshape(1, 2 * d))

    full = lambda shape: pl.BlockSpec(shape, lambda i: (0, 0))
    out = pl.pallas_call(
        _crsn_body,
        grid=(nb,),
        in_specs=[
            pl.BlockSpec((_BLK, 1), lambda i: (i, 0)),
            full((_NSYM, d)), full((_NSYM, d)),
            full((d, d)), full((d, d)),
            full((1, 2 * d)), full((1, 2 * d)), full((1, 2 * d)),
            full((8, 2 * d)), full((1, 1)),
            full((_NSYM, 2 * d)), full((_NSYM, _NSYM)),
            full((_NSYM, 2 * d)), full((1, _NSYM)),
        ],
        out_specs=[
            pl.BlockSpec((_BLK, _NSYM), lambda i: (i, 0)),
            pl.BlockSpec((_BLK, 2 * d), lambda i: (i, 0)),
            pl.BlockSpec((_BLK, 1), lambda i: (i, 0)),
            pl.BlockSpec((1, 2), lambda i: (0, 0)),
        ],
        out_shape=[
            jax.ShapeDtypeStruct((batch, _NSYM), jnp.float32),
            jax.ShapeDtypeStruct((batch, 2 * d), jnp.float32),
            jax.ShapeDtypeStruct((batch, 1), jnp.int32),
            jax.ShapeDtypeStruct((1, 2), jnp.float32),
        ],
        interpret=interpret,
    )(x2, em_p, ep_p, Wr, Wi, lns2, lnb2, mb2, hw8,
      halt_b.reshape(1, 1).astype(jnp.float32), codebook, adj, dw_p, db_p)

    logits_p, feats, sym2, stats = out
    logits = logits_p[:, :dec_W.shape[0]]
    z_accum = jax.lax.complex(feats[:, :d], feats[:, d:])
    sym = sym2[:, 0]
    ponder = stats[0, 0] / batch
    vq_total = stats[0, 1] * (1.25 / (batch * 2 * d))
    return (logits, z_accum, sym, ponder, vq_total)


def kernel(x, emb_mag, emb_phase, Wr, Wi, ln_scale, ln_shift, mod_bias,
           halt_W, halt_b, codebook, adj, dec_W, dec_b):
    return _run(x, emb_mag, emb_phase, Wr, Wi, ln_scale, ln_shift, mod_bias,
                halt_W, halt_b, codebook, adj, dec_W, dec_b)
